# baseline (speedup 1.0000x reference)
import functools

import jax
import jax.numpy as jnp
from jax import lax
from jax.experimental import pallas as pl
from jax.experimental.pallas import tpu as pltpu
from jax.experimental.pallas import tpu_sc as plsc

_D = 256
_NBINS = 256
_PREC = lax.Precision.DEFAULT


def _dot(a, b):
    return jnp.dot(a, b, preferred_element_type=jnp.float32, precision=_PREC)


def _conv3(h, w, b):
    p0 = _dot(h, w[0])
    p1 = _dot(h, w[1])
    p2 = _dot(h, w[2])
    z = jnp.zeros((1, p0.shape[1]), jnp.float32)
    prev = jnp.concatenate([z, p0[:-1]], axis=0)
    nxt = jnp.concatenate([p2[1:], z], axis=0)
    return prev + p1 + nxt + b


def _layernorm(h, g, b):
    m = jnp.mean(h, axis=-1, keepdims=True)
    v = jnp.mean((h - m) ** 2, axis=-1, keepdims=True)
    return (h - m) / jnp.sqrt(v + 1e-5) * g + b


def _predictor_body(h, w1, b1, g1, be1, w2, b2, g2, be2, wl, bl):
    h = jnp.maximum(_conv3(h, w1, b1), 0.0)
    h = _layernorm(h, g1, be1)
    h = jnp.maximum(_conv3(h, w2, b2), 0.0)
    h = _layernorm(h, g2, be2)
    return jnp.maximum(_dot(h, wl) + bl, 0.0)


def _tc_kernel(x_ref, al_ref,
               dw1, db1, dg1, dbe1, dw2, db2, dg2, dbe2, dwl, dbl,
               pw1, pb1, pg1, pbe1, pw2, pb2, pg2, pbe2, pwl, pbl,
               ew1, eb1, eg1, ebe1, ew2, eb2, eg2, ebe2, ewl, ebl,
               out_ref, dur_ref, pp_ref, lp_ref, ep_ref, mm_ref):
    x = x_ref[0]
    dur_ref[0] = _predictor_body(x, dw1[...], db1[...], dg1[...], dbe1[...],
                                 dw2[...], db2[...], dg2[...], dbe2[...],
                                 dwl[...], dbl[...])
    out = _dot(al_ref[0], x)
    out_ref[0] = out
    pp = _predictor_body(out, pw1[...], pb1[...], pg1[...], pbe1[...],
                         pw2[...], pb2[...], pg2[...], pbe2[...],
                         pwl[...], pbl[...])
    pp_ref[0] = pp
    lp = jnp.log(pp + 1e-12)
    lp_ref[0] = lp
    ep = _predictor_body(out, ew1[...], eb1[...], eg1[...], ebe1[...],
                         ew2[...], eb2[...], eg2[...], ebe2[...],
                         ewl[...], ebl[...])
    ep_ref[0] = ep
    b = pl.program_id(0)
    sub = lax.broadcasted_iota(jnp.int32, (8, 128), 0)
    lane = lax.broadcasted_iota(jnp.int32, (8, 128), 1)
    val = jnp.where(sub == 0, jnp.min(lp),
                    jnp.where(sub == 1, jnp.max(lp),
                              jnp.where(sub == 2, jnp.min(ep), jnp.max(ep))))
    mm_ref[...] = jnp.where(lane == b, val, mm_ref[...])


def _flatten_params(p):
    return (p['w1'], p['b1'].reshape(1, -1), p['g1'].reshape(1, -1),
            p['be1'].reshape(1, -1), p['w2'], p['b2'].reshape(1, -1),
            p['g2'].reshape(1, -1), p['be2'].reshape(1, -1),
            p['wl'], p['bl'].reshape(1, 1))


def _run_tc(x, alignment, dur_params, pitch_params, energy_params):
    B, T_src, D = x.shape
    T_mel = alignment.shape[1]

    w_args = (_flatten_params(dur_params) + _flatten_params(pitch_params)
              + _flatten_params(energy_params))

    def w_spec(a):
        return pl.BlockSpec(a.shape, lambda b: (0,) * a.ndim)

    in_specs = [
        pl.BlockSpec((1, T_src, D), lambda b: (b, 0, 0)),
        pl.BlockSpec((1, T_mel, T_src), lambda b: (b, 0, 0)),
    ] + [w_spec(a) for a in w_args]

    out_shapes = [
        jax.ShapeDtypeStruct((B, T_mel, D), jnp.float32),
        jax.ShapeDtypeStruct((B, T_src, 1), jnp.float32),
        jax.ShapeDtypeStruct((B, T_mel, 1), jnp.float32),
        jax.ShapeDtypeStruct((B, T_mel, 1), jnp.float32),
        jax.ShapeDtypeStruct((B, T_mel, 1), jnp.float32),
        jax.ShapeDtypeStruct((8, 128), jnp.float32),
    ]
    out_specs = [
        pl.BlockSpec((1, T_mel, D), lambda b: (b, 0, 0)),
        pl.BlockSpec((1, T_src, 1), lambda b: (b, 0, 0)),
        pl.BlockSpec((1, T_mel, 1), lambda b: (b, 0, 0)),
        pl.BlockSpec((1, T_mel, 1), lambda b: (b, 0, 0)),
        pl.BlockSpec((1, T_mel, 1), lambda b: (b, 0, 0)),
        pl.BlockSpec((8, 128), lambda b: (0, 0)),
    ]
    return pl.pallas_call(
        _tc_kernel,
        grid=(B,),
        in_specs=in_specs,
        out_specs=out_specs,
        out_shape=out_shapes,
    )(x, alignment, *w_args)


_NC = 2
_NS = 16
_L = 16
_CH = 128


_CW = _D // 2
_RPT = 2048


def _sc_kernel(out_raw, lp, ep, minmax, pemb, eemb, out_final,
               lp_v, ep_v, pidx_v, eidx_v, ptbl, etbl, outv0, outv1, mm_v,
               semi0, semi1, semo0, semo1):
    wid = lax.axis_index("s") * _NC + lax.axis_index("c")
    rb = wid // 2
    chalf = wid % 2
    row0 = rb * _RPT
    col0 = chalf * _CW

    pltpu.sync_copy(minmax, mm_v)
    pltpu.sync_copy(pemb.at[chalf], ptbl)
    pltpu.sync_copy(eemb.at[chalf], etbl)
    pltpu.sync_copy(lp.at[pl.ds(row0, _RPT)], lp_v)
    pltpu.sync_copy(ep.at[pl.ds(row0, _RPT)], ep_v)

    nb = jnp.full((_L,), float(_NBINS), jnp.float32)
    lmin = jnp.full((_L,), jnp.min(mm_v[0, pl.ds(0, _L)]))
    lmax = jnp.full((_L,), jnp.max(mm_v[1, pl.ds(0, _L)]))
    emin = jnp.full((_L,), jnp.min(mm_v[2, pl.ds(0, _L)]))
    emax = jnp.full((_L,), jnp.max(mm_v[3, pl.ds(0, _L)]))
    psc = nb / (lmax - lmin)
    esc = nb / (emax - emin)

    @plsc.parallel_loop(0, _RPT // _L, unroll=4)
    def _ib(i):
        s = pl.ds(i * _L, _L)
        q = (lp_v[s] - lmin) * psc
        pidx_v[s] = jnp.clip(q.astype(jnp.int32), 0, _NBINS - 1) * _CW
        q2 = (ep_v[s] - emin) * esc
        eidx_v[s] = jnp.clip(q2.astype(jnp.int32), 0, _NBINS - 1) * _CW

    iota = lax.iota(jnp.int32, _L)
    coloffs = [iota + (j * _L) for j in range(_CW // _L)]
    n_chunks = _RPT // _CH
    bufs = (outv0, outv1)
    semis = (semi0, semi1)
    semos = (semo0, semo1)

    def in_slice(k):
        return out_raw.at[pl.ds(row0 + k * _CH, _CH), pl.ds(col0, _CW)]

    def out_slice(k):
        return out_final.at[pl.ds(row0 + k * _CH, _CH), pl.ds(col0, _CW)]

    in_cp = [None] * n_chunks
    out_cp = [None] * n_chunks
    in_cp[0] = pltpu.async_copy(in_slice(0), bufs[0], semis[0])
    for k in range(n_chunks):
        b = k % 2
        in_cp[k].wait()
        if k + 1 < n_chunks:
            if k >= 1:
                out_cp[k - 1].wait()
            in_cp[k + 1] = pltpu.async_copy(in_slice(k + 1), bufs[1 - b],
                                            semis[1 - b])
        outv = bufs[b]

        @plsc.parallel_loop(0, _CH, unroll=2)
        def _rb(r):
            rs = jnp.full((_L,), k * _CH, jnp.int32) + r
            pb = plsc.load_gather(pidx_v, [rs])
            eb = plsc.load_gather(eidx_v, [rs])
            for j in range(_CW // _L):
                a = plsc.load_gather(ptbl, [pb + coloffs[j]])
                e = plsc.load_gather(etbl, [eb + coloffs[j]])
                plsc.addupdate(outv.at[r, pl.ds(j * _L, _L)], a + e)
        out_cp[k] = pltpu.async_copy(outv, out_slice(k), semos[b])
    out_cp[n_chunks - 2].wait()
    out_cp[n_chunks - 1].wait()


def _run_sc(out_raw, lp, ep, minmax, pitch_emb, energy_emb):
    rows = out_raw.shape[0]
    mesh = plsc.VectorSubcoreMesh(core_axis_name="c", subcore_axis_name="s",
                                  num_cores=_NC, num_subcores=_NS)
    f = pl.kernel(
        _sc_kernel,
        out_type=jax.ShapeDtypeStruct((rows, _D), jnp.float32),
        mesh=mesh,
        scratch_types=[
            pltpu.VMEM((_RPT,), jnp.float32),
            pltpu.VMEM((_RPT,), jnp.float32),
            pltpu.VMEM((_RPT,), jnp.int32),
            pltpu.VMEM((_RPT,), jnp.int32),
            pltpu.VMEM((_NBINS * _CW,), jnp.float32),
            pltpu.VMEM((_NBINS * _CW,), jnp.float32),
            pltpu.VMEM((_CH, _CW), jnp.float32),
            pltpu.VMEM((_CH, _CW), jnp.float32),
            pltpu.VMEM((8, 128), jnp.float32),
            pltpu.SemaphoreType.DMA,
            pltpu.SemaphoreType.DMA,
            pltpu.SemaphoreType.DMA,
            pltpu.SemaphoreType.DMA,
        ],
        compiler_params=pltpu.CompilerParams(needs_layout_passes=False),
    )
    return f(out_raw, lp, ep, minmax, pitch_emb, energy_emb)


def kernel(x, alignment, dur_params, pitch_params, energy_params,
           pitch_emb, energy_emb):
    B, T_src, D = x.shape
    T_mel = alignment.shape[1]

    out_raw, dur, pp, lp, ep, minmax = _run_tc(x, alignment, dur_params,
                                               pitch_params, energy_params)

    lp_f = lp.reshape(B * T_mel)
    ep_f = ep.reshape(B * T_mel)
    ptab = jnp.stack([pitch_emb[:, :_D // 2].reshape(-1),
                      pitch_emb[:, _D // 2:].reshape(-1)])
    etab = jnp.stack([energy_emb[:, :_D // 2].reshape(-1),
                      energy_emb[:, _D // 2:].reshape(-1)])
    out = _run_sc(out_raw.reshape(B * T_mel, D), lp_f, ep_f, minmax,
                  ptab, etab)

    return (out.reshape(B, T_mel, D), dur.reshape(B, T_src),
            pp.reshape(B, T_mel), ep.reshape(B, T_mel))


# speedup vs baseline: 1.0742x; 1.0742x over previous
import functools

import jax
import jax.numpy as jnp
from jax import lax
from jax.experimental import pallas as pl
from jax.experimental.pallas import tpu as pltpu
from jax.experimental.pallas import tpu_sc as plsc

_D = 256
_NBINS = 256
_PREC = lax.Precision.DEFAULT


def _dot(a, b):
    return jnp.dot(a, b, preferred_element_type=jnp.float32, precision=_PREC)


def _conv3(h, w, b):
    p0 = _dot(h, w[0])
    p1 = _dot(h, w[1])
    p2 = _dot(h, w[2])
    z = jnp.zeros((1, p0.shape[1]), jnp.float32)
    prev = jnp.concatenate([z, p0[:-1]], axis=0)
    nxt = jnp.concatenate([p2[1:], z], axis=0)
    return prev + p1 + nxt + b


def _layernorm(h, g, b):
    m = jnp.mean(h, axis=-1, keepdims=True)
    v = jnp.mean((h - m) ** 2, axis=-1, keepdims=True)
    return (h - m) / jnp.sqrt(v + 1e-5) * g + b


def _predictor_body(h, w1, b1, g1, be1, w2, b2, g2, be2, wl, bl):
    h = jnp.maximum(_conv3(h, w1, b1), 0.0)
    h = _layernorm(h, g1, be1)
    h = jnp.maximum(_conv3(h, w2, b2), 0.0)
    h = _layernorm(h, g2, be2)
    out = lax.dot_general(wl, h, (((0,), (1,)), ((), ())),
                          preferred_element_type=jnp.float32,
                          precision=_PREC)
    return jnp.maximum(out + bl, 0.0)


def _tc_kernel(x_ref, al_ref,
               dw1, db1, dg1, dbe1, dw2, db2, dg2, dbe2, dwl, dbl,
               pw1, pb1, pg1, pbe1, pw2, pb2, pg2, pbe2, pwl, pbl,
               ew1, eb1, eg1, ebe1, ew2, eb2, eg2, ebe2, ewl, ebl,
               out_ref, dur_ref, pp_ref, lp_ref, ep_ref, mm_ref):
    x = x_ref[0]
    dur_ref[0] = _predictor_body(x, dw1[...], db1[...], dg1[...], dbe1[...],
                                 dw2[...], db2[...], dg2[...], dbe2[...],
                                 dwl[...], dbl[...])
    out = _dot(al_ref[0], x)
    out_ref[0] = out
    pp = _predictor_body(out, pw1[...], pb1[...], pg1[...], pbe1[...],
                         pw2[...], pb2[...], pg2[...], pbe2[...],
                         pwl[...], pbl[...])
    pp_ref[0] = pp
    lp = jnp.log(pp + 1e-12)
    lp_ref[0] = lp
    ep = _predictor_body(out, ew1[...], eb1[...], eg1[...], ebe1[...],
                         ew2[...], eb2[...], eg2[...], ebe2[...],
                         ewl[...], ebl[...])
    ep_ref[0] = ep
    b = pl.program_id(0)
    sub = lax.broadcasted_iota(jnp.int32, (8, 128), 0)
    lane = lax.broadcasted_iota(jnp.int32, (8, 128), 1)
    val = jnp.where(sub == 0, jnp.min(lp),
                    jnp.where(sub == 1, jnp.max(lp),
                              jnp.where(sub == 2, jnp.min(ep), jnp.max(ep))))
    mm_ref[...] = jnp.where(lane == b, val, mm_ref[...])


def _flatten_params(p):
    return (p['w1'], p['b1'].reshape(1, -1), p['g1'].reshape(1, -1),
            p['be1'].reshape(1, -1), p['w2'], p['b2'].reshape(1, -1),
            p['g2'].reshape(1, -1), p['be2'].reshape(1, -1),
            p['wl'], p['bl'].reshape(1, 1))


def _run_tc(x, alignment, dur_params, pitch_params, energy_params):
    B, T_src, D = x.shape
    T_mel = alignment.shape[1]

    w_args = (_flatten_params(dur_params) + _flatten_params(pitch_params)
              + _flatten_params(energy_params))

    def w_spec(a):
        return pl.BlockSpec(a.shape, lambda b: (0,) * a.ndim)

    in_specs = [
        pl.BlockSpec((1, T_src, D), lambda b: (b, 0, 0)),
        pl.BlockSpec((1, T_mel, T_src), lambda b: (b, 0, 0)),
    ] + [w_spec(a) for a in w_args]

    out_shapes = [
        jax.ShapeDtypeStruct((B, T_mel, D), jnp.float32),
        jax.ShapeDtypeStruct((B, 1, T_src), jnp.float32),
        jax.ShapeDtypeStruct((B, 1, T_mel), jnp.float32),
        jax.ShapeDtypeStruct((B, 1, T_mel), jnp.float32),
        jax.ShapeDtypeStruct((B, 1, T_mel), jnp.float32),
        jax.ShapeDtypeStruct((8, 128), jnp.float32),
    ]
    out_specs = [
        pl.BlockSpec((1, T_mel, D), lambda b: (b, 0, 0)),
        pl.BlockSpec((1, 1, T_src), lambda b: (b, 0, 0)),
        pl.BlockSpec((1, 1, T_mel), lambda b: (b, 0, 0)),
        pl.BlockSpec((1, 1, T_mel), lambda b: (b, 0, 0)),
        pl.BlockSpec((1, 1, T_mel), lambda b: (b, 0, 0)),
        pl.BlockSpec((8, 128), lambda b: (0, 0)),
    ]
    return pl.pallas_call(
        _tc_kernel,
        grid=(B,),
        in_specs=in_specs,
        out_specs=out_specs,
        out_shape=out_shapes,
    )(x, alignment, *w_args)


_NC = 2
_NS = 16
_L = 16
_CH = 128


_CW = _D // 2
_RPT = 2048


def _sc_kernel(out_raw, lp, ep, minmax, pemb, eemb, out_final,
               lp_v, ep_v, pidx_v, eidx_v, ptbl, etbl, outv0, outv1, mm_v,
               semi0, semi1, semo0, semo1):
    wid = lax.axis_index("s") * _NC + lax.axis_index("c")
    rb = wid // 2
    chalf = wid % 2
    row0 = rb * _RPT
    col0 = chalf * _CW

    pltpu.sync_copy(minmax, mm_v)
    pltpu.sync_copy(pemb.at[chalf], ptbl)
    pltpu.sync_copy(eemb.at[chalf], etbl)
    pltpu.sync_copy(lp.at[pl.ds(row0, _RPT)], lp_v)
    pltpu.sync_copy(ep.at[pl.ds(row0, _RPT)], ep_v)

    nb = jnp.full((_L,), float(_NBINS), jnp.float32)
    lmin = jnp.full((_L,), jnp.min(mm_v[0, pl.ds(0, _L)]))
    lmax = jnp.full((_L,), jnp.max(mm_v[1, pl.ds(0, _L)]))
    emin = jnp.full((_L,), jnp.min(mm_v[2, pl.ds(0, _L)]))
    emax = jnp.full((_L,), jnp.max(mm_v[3, pl.ds(0, _L)]))
    psc = nb / (lmax - lmin)
    esc = nb / (emax - emin)

    @plsc.parallel_loop(0, _RPT // _L, unroll=4)
    def _ib(i):
        s = pl.ds(i * _L, _L)
        q = (lp_v[s] - lmin) * psc
        pidx_v[s] = jnp.clip(q.astype(jnp.int32), 0, _NBINS - 1) * _CW
        q2 = (ep_v[s] - emin) * esc
        eidx_v[s] = jnp.clip(q2.astype(jnp.int32), 0, _NBINS - 1) * _CW

    iota = lax.iota(jnp.int32, _L)
    coloffs = [iota + (j * _L) for j in range(_CW // _L)]
    n_chunks = _RPT // _CH
    bufs = (outv0, outv1)
    semis = (semi0, semi1)
    semos = (semo0, semo1)

    def in_slice(k):
        return out_raw.at[pl.ds(row0 + k * _CH, _CH), pl.ds(col0, _CW)]

    def out_slice(k):
        return out_final.at[pl.ds(row0 + k * _CH, _CH), pl.ds(col0, _CW)]

    in_cp = [None] * n_chunks
    out_cp = [None] * n_chunks
    in_cp[0] = pltpu.async_copy(in_slice(0), bufs[0], semis[0])
    for k in range(n_chunks):
        b = k % 2
        in_cp[k].wait()
        if k + 1 < n_chunks:
            if k >= 1:
                out_cp[k - 1].wait()
            in_cp[k + 1] = pltpu.async_copy(in_slice(k + 1), bufs[1 - b],
                                            semis[1 - b])
        outv = bufs[b]

        @plsc.parallel_loop(0, _CH, unroll=2)
        def _rb(r):
            rs = jnp.full((_L,), k * _CH, jnp.int32) + r
            pb = plsc.load_gather(pidx_v, [rs])
            eb = plsc.load_gather(eidx_v, [rs])
            for j in range(_CW // _L):
                a = plsc.load_gather(ptbl, [pb + coloffs[j]])
                e = plsc.load_gather(etbl, [eb + coloffs[j]])
                plsc.addupdate(outv.at[r, pl.ds(j * _L, _L)], a + e)
        out_cp[k] = pltpu.async_copy(outv, out_slice(k), semos[b])
    out_cp[n_chunks - 2].wait()
    out_cp[n_chunks - 1].wait()


def _run_sc(out_raw, lp, ep, minmax, pitch_emb, energy_emb):
    rows = out_raw.shape[0]
    mesh = plsc.VectorSubcoreMesh(core_axis_name="c", subcore_axis_name="s",
                                  num_cores=_NC, num_subcores=_NS)
    f = pl.kernel(
        _sc_kernel,
        out_type=jax.ShapeDtypeStruct((rows, _D), jnp.float32),
        mesh=mesh,
        scratch_types=[
            pltpu.VMEM((_RPT,), jnp.float32),
            pltpu.VMEM((_RPT,), jnp.float32),
            pltpu.VMEM((_RPT,), jnp.int32),
            pltpu.VMEM((_RPT,), jnp.int32),
            pltpu.VMEM((_NBINS * _CW,), jnp.float32),
            pltpu.VMEM((_NBINS * _CW,), jnp.float32),
            pltpu.VMEM((_CH, _CW), jnp.float32),
            pltpu.VMEM((_CH, _CW), jnp.float32),
            pltpu.VMEM((8, 128), jnp.float32),
            pltpu.SemaphoreType.DMA,
            pltpu.SemaphoreType.DMA,
            pltpu.SemaphoreType.DMA,
            pltpu.SemaphoreType.DMA,
        ],
        compiler_params=pltpu.CompilerParams(needs_layout_passes=False),
    )
    return f(out_raw, lp, ep, minmax, pitch_emb, energy_emb)


def kernel(x, alignment, dur_params, pitch_params, energy_params,
           pitch_emb, energy_emb):
    B, T_src, D = x.shape
    T_mel = alignment.shape[1]

    out_raw, dur, pp, lp, ep, minmax = _run_tc(x, alignment, dur_params,
                                               pitch_params, energy_params)

    lp_f = lp.reshape(B * T_mel)
    ep_f = ep.reshape(B * T_mel)
    ptab = jnp.stack([pitch_emb[:, :_D // 2].reshape(-1),
                      pitch_emb[:, _D // 2:].reshape(-1)])
    etab = jnp.stack([energy_emb[:, :_D // 2].reshape(-1),
                      energy_emb[:, _D // 2:].reshape(-1)])
    out = _run_sc(out_raw.reshape(B * T_mel, D), lp_f, ep_f, minmax,
                  ptab, etab)

    return (out.reshape(B, T_mel, D), dur.reshape(B, T_src),
            pp.reshape(B, T_mel), ep.reshape(B, T_mel))
